# baseline (speedup 1.0000x reference)
import jax
import jax.numpy as jnp
from jax.experimental import pallas as pl
from jax.experimental.pallas import tpu as pltpu

_N = 4096
_CHUNK = 128
_NCH = _N // _CHUNK
_OUTER_STEPS = (256, 512, 1024, 2048, 4096)


def _chunk_transform(cos_t, sin_t):
    z = jnp.eye(_CHUNK, dtype=jnp.complex64)
    step = 2
    while step <= _CHUNK:
        half = step // 2
        stride = _N // step
        tw = jax.lax.complex(cos_t[::stride], sin_t[::stride])
        xr = z.reshape(_CHUNK, _CHUNK // step, step)
        a = xr[:, :, :half]
        b = xr[:, :, half:]
        t = tw * b
        z = jnp.concatenate([a + t, a - t], axis=-1).reshape(_CHUNK, _CHUNK)
        step *= 2
    return z


def _build_factors(w, deinterleave):
    ang = (-2.0 * jnp.pi / _N) * jnp.arange(_N // 2, dtype=jnp.float32) * w
    cos_t, sin_t = jnp.cos(ang), jnp.sin(ang)
    t = _chunk_transform(cos_t, sin_t)
    tr, ti = jnp.real(t), jnp.imag(t)
    rows_a = jnp.concatenate([tr, ti], axis=1)
    rows_b = jnp.concatenate([-ti, tr], axis=1)
    if deinterleave:
        m = jnp.stack([rows_a, rows_b], axis=1).reshape(256, 256)
    else:
        m = jnp.concatenate([rows_a, rows_b], axis=0)
    rows = []
    for step in _OUTER_STEPS:
        half = step // 2
        stride = _N // step
        rows.append(jnp.pad(cos_t[::stride], (0, 2048 - half)))
        rows.append(jnp.pad(sin_t[::stride], (0, 2048 - half)))
    rows.extend([jnp.zeros((2048,), jnp.float32)] * 6)
    return m, jnp.stack(rows)


def _pair_t(h_ref, tw_ref, i, c1, c2, ol):
    to = 128 * ol
    twr = tw_ref[2 * i:2 * i + 1, to:to + 128]
    twi = tw_ref[2 * i + 1:2 * i + 2, to:to + 128]
    ar = h_ref[:, c1:c1 + 128]
    ai = h_ref[:, c1 + 128:c1 + 256]
    br = h_ref[:, c2:c2 + 128]
    bi = h_ref[:, c2 + 128:c2 + 256]
    tre = twr * br - twi * bi
    tim = twr * bi + twi * br
    return ar, ai, tre, tim


def _outer_stages(h_ref, tw_ref, relu_last=False, head_only=False):
    for i, step in enumerate(_OUTER_STEPS):
        if head_only and step == 4096:
            return
        cpg = step // _CHUNK
        hc = cpg // 2
        for g in range(_N // step):
            for ol in range(hc):
                c1 = (g * cpg + ol) * 256
                c2 = c1 + hc * 256
                ar, ai, tre, tim = _pair_t(h_ref, tw_ref, i, c1, c2, ol)
                if relu_last and step == 4096:
                    h_ref[:, c1:c1 + 128] = jnp.maximum(ar + tre, 0.0)
                    h_ref[:, c1 + 128:c1 + 256] = jnp.maximum(ai + tim, 0.0)
                    h_ref[:, c2:c2 + 128] = jnp.maximum(ar - tre, 0.0)
                    h_ref[:, c2 + 128:c2 + 256] = jnp.maximum(ai - tim, 0.0)
                else:
                    h_ref[:, c1:c1 + 128] = ar + tre
                    h_ref[:, c1 + 128:c1 + 256] = ai + tim
                    if head_only and step == 2048:
                        continue
                    h_ref[:, c2:c2 + 128] = ar - tre
                    h_ref[:, c2 + 128:c2 + 256] = ai - tim


def _fwd_kernel(x_ref, m1_ref, m2_ref, tw1_ref, tw2_ref, p_ref, out_ref,
                h_ref):
    m1 = m1_ref[...]
    for l in range(_NCH):
        src = (l ^ (_NCH // 2)) * 256
        h_ref[:, l * 256:(l + 1) * 256] = jnp.dot(
            x_ref[:, src:src + 256], m1, preferred_element_type=jnp.float32)
    _outer_stages(h_ref, tw1_ref, relu_last=True)
    m2 = m2_ref[...]
    for l in range(_NCH // 2):
        a = h_ref[:, l * 256:(l + 1) * 256]
        b = h_ref[:, (l + 16) * 256:(l + 17) * 256]
        h_ref[:, l * 256:(l + 1) * 256] = jnp.dot(
            b, m2, preferred_element_type=jnp.float32)
        h_ref[:, (l + 16) * 256:(l + 17) * 256] = jnp.dot(
            a, m2, preferred_element_type=jnp.float32)
    _outer_stages(h_ref, tw2_ref, head_only=True)
    p = p_ref[...]
    for l in range(8):
        ar, ai, tre, tim = _pair_t(h_ref, tw2_ref, 4, l * 256,
                                   (l + 16) * 256, l)
        v = jnp.concatenate([ar + tre, ai + tim], axis=1)
        out_ref[:, l * 256:(l + 1) * 256] = jnp.dot(
            v, p, preferred_element_type=jnp.float32)


@jax.jit
def kernel(x, w1, w2):
    b = x.shape[0]
    bblk = 128 if b % 128 == 0 else b
    m1, tw1 = _build_factors(w1, deinterleave=True)
    m2, tw2 = _build_factors(w2, deinterleave=False)
    j = jnp.arange(256)
    perm_cols = jnp.where(j < 128, 2 * j, 2 * (j - 128) + 1)
    p = (perm_cols[:, None] == j[None, :]).astype(jnp.float32)
    return pl.pallas_call(
        _fwd_kernel,
        grid=(b // bblk,),
        in_specs=[
            pl.BlockSpec((bblk, 2 * _N), lambda i: (i, 0)),
            pl.BlockSpec((256, 256), lambda i: (0, 0)),
            pl.BlockSpec((256, 256), lambda i: (0, 0)),
            pl.BlockSpec((16, 2048), lambda i: (0, 0)),
            pl.BlockSpec((16, 2048), lambda i: (0, 0)),
            pl.BlockSpec((256, 256), lambda i: (0, 0)),
        ],
        out_specs=pl.BlockSpec((bblk, 2048), lambda i: (i, 0)),
        out_shape=jax.ShapeDtypeStruct((b, 2048), jnp.float32),
        scratch_shapes=[pltpu.VMEM((bblk, 2 * _N), jnp.float32)],
        compiler_params=pltpu.CompilerParams(
            dimension_semantics=("parallel",),
            vmem_limit_bytes=60 * 1024 * 1024,
        ),
    )(x, m1, m2, tw1, tw2, p)


# speedup vs baseline: 1.1250x; 1.1250x over previous
import jax
import jax.numpy as jnp
from jax.experimental import pallas as pl
from jax.experimental.pallas import tpu as pltpu

_N = 4096
_CHUNK = 128
_NCH = _N // _CHUNK
_OUTER_STEPS = (256, 512, 1024, 2048, 4096)


def _build_factors(w1, w2):
    w = jnp.stack([w1, w2])
    ang = (-2.0 * jnp.pi / _N) * jnp.arange(_N // 2, dtype=jnp.float32) * w
    cos_t, sin_t = jnp.cos(ang), jnp.sin(ang)
    z = jnp.broadcast_to(jnp.eye(_CHUNK, dtype=jnp.complex64),
                         (2, _CHUNK, _CHUNK))
    step = 2
    while step <= _CHUNK:
        half = step // 2
        stride = _N // step
        tw = jax.lax.complex(cos_t[:, ::stride], sin_t[:, ::stride])
        xr = z.reshape(2, _CHUNK, _CHUNK // step, step)
        a = xr[:, :, :, :half]
        b = xr[:, :, :, half:]
        t = tw[:, None, None, :] * b
        z = jnp.concatenate([a + t, a - t], axis=-1).reshape(
            2, _CHUNK, _CHUNK)
        step *= 2
    tr, ti = jnp.real(z), jnp.imag(z)
    rows_a = jnp.concatenate([tr, ti], axis=2)
    rows_b = jnp.concatenate([-ti, tr], axis=2)
    m1 = jnp.stack([rows_a[0], rows_b[0]], axis=1).reshape(256, 256)
    m2 = jnp.concatenate([rows_a[1], rows_b[1]], axis=0)
    rows = []
    for step in _OUTER_STEPS:
        half = step // 2
        stride = _N // step
        rows.append(jnp.pad(cos_t[:, ::stride], ((0, 0), (0, 2048 - half))))
        rows.append(jnp.pad(sin_t[:, ::stride], ((0, 0), (0, 2048 - half))))
    rows.extend([jnp.zeros((2, 2048), jnp.float32)] * 6)
    tw_pack = jnp.stack(rows).transpose(1, 0, 2)
    return m1, m2, tw_pack[0], tw_pack[1]


def _pair_t(h_ref, tw_ref, i, c1, c2, ol):
    to = 128 * ol
    twr = tw_ref[2 * i:2 * i + 1, to:to + 128]
    twi = tw_ref[2 * i + 1:2 * i + 2, to:to + 128]
    ar = h_ref[:, c1:c1 + 128]
    ai = h_ref[:, c1 + 128:c1 + 256]
    br = h_ref[:, c2:c2 + 128]
    bi = h_ref[:, c2 + 128:c2 + 256]
    tre = twr * br - twi * bi
    tim = twr * bi + twi * br
    return ar, ai, tre, tim


def _outer_stages(h_ref, tw_ref, relu_last=False, head_only=False):
    for i, step in enumerate(_OUTER_STEPS):
        if head_only and step == 4096:
            return
        cpg = step // _CHUNK
        hc = cpg // 2
        for g in range(_N // step):
            for ol in range(hc):
                c1 = (g * cpg + ol) * 256
                c2 = c1 + hc * 256
                ar, ai, tre, tim = _pair_t(h_ref, tw_ref, i, c1, c2, ol)
                if relu_last and step == 4096:
                    h_ref[:, c1:c1 + 128] = jnp.maximum(ar + tre, 0.0)
                    h_ref[:, c1 + 128:c1 + 256] = jnp.maximum(ai + tim, 0.0)
                    h_ref[:, c2:c2 + 128] = jnp.maximum(ar - tre, 0.0)
                    h_ref[:, c2 + 128:c2 + 256] = jnp.maximum(ai - tim, 0.0)
                else:
                    h_ref[:, c1:c1 + 128] = ar + tre
                    h_ref[:, c1 + 128:c1 + 256] = ai + tim
                    if head_only and step == 2048:
                        continue
                    h_ref[:, c2:c2 + 128] = ar - tre
                    h_ref[:, c2 + 128:c2 + 256] = ai - tim


def _fwd_kernel(x_ref, m1_ref, m2_ref, tw1_ref, tw2_ref, p_ref, out_ref,
                h_ref):
    m1 = m1_ref[...]
    for l in range(_NCH):
        src = (l ^ (_NCH // 2)) * 256
        h_ref[:, l * 256:(l + 1) * 256] = jnp.dot(
            x_ref[:, src:src + 256], m1, preferred_element_type=jnp.float32)
    _outer_stages(h_ref, tw1_ref, relu_last=True)
    m2 = m2_ref[...]
    for l in range(_NCH // 2):
        a = h_ref[:, l * 256:(l + 1) * 256]
        b = h_ref[:, (l + 16) * 256:(l + 17) * 256]
        h_ref[:, l * 256:(l + 1) * 256] = jnp.dot(
            b, m2, preferred_element_type=jnp.float32)
        h_ref[:, (l + 16) * 256:(l + 17) * 256] = jnp.dot(
            a, m2, preferred_element_type=jnp.float32)
    _outer_stages(h_ref, tw2_ref, head_only=True)
    p = p_ref[...]
    for l in range(8):
        ar, ai, tre, tim = _pair_t(h_ref, tw2_ref, 4, l * 256,
                                   (l + 16) * 256, l)
        v = jnp.concatenate([ar + tre, ai + tim], axis=1)
        out_ref[:, l * 256:(l + 1) * 256] = jnp.dot(
            v, p, preferred_element_type=jnp.float32)


@jax.jit
def kernel(x, w1, w2):
    b = x.shape[0]
    bblk = 256 if b % 256 == 0 else b
    m1, m2, tw1, tw2 = _build_factors(w1, w2)
    j = jnp.arange(256)
    perm_cols = jnp.where(j < 128, 2 * j, 2 * (j - 128) + 1)
    p = (perm_cols[:, None] == j[None, :]).astype(jnp.float32)
    return pl.pallas_call(
        _fwd_kernel,
        grid=(b // bblk,),
        in_specs=[
            pl.BlockSpec((bblk, 2 * _N), lambda i: (i, 0)),
            pl.BlockSpec((256, 256), lambda i: (0, 0)),
            pl.BlockSpec((256, 256), lambda i: (0, 0)),
            pl.BlockSpec((16, 2048), lambda i: (0, 0)),
            pl.BlockSpec((16, 2048), lambda i: (0, 0)),
            pl.BlockSpec((256, 256), lambda i: (0, 0)),
        ],
        out_specs=pl.BlockSpec((bblk, 2048), lambda i: (i, 0)),
        out_shape=jax.ShapeDtypeStruct((b, 2048), jnp.float32),
        scratch_shapes=[pltpu.VMEM((bblk, 2 * _N), jnp.float32)],
        compiler_params=pltpu.CompilerParams(
            dimension_semantics=("parallel",),
            vmem_limit_bytes=60 * 1024 * 1024,
        ),
    )(x, m1, m2, tw1, tw2, p)


# speedup vs baseline: 1.1351x; 1.0090x over previous
import jax
import jax.numpy as jnp
from jax.experimental import pallas as pl
from jax.experimental.pallas import tpu as pltpu

_N = 4096
_CHUNK = 128
_NCH = _N // _CHUNK
_OUTER_STEPS = (256, 512, 1024, 2048, 4096)


def _build_factors(w1, w2):
    w = jnp.stack([w1, w2])
    ang = (-2.0 * jnp.pi / _N) * jnp.arange(_N // 2, dtype=jnp.float32) * w
    cos_t, sin_t = jnp.cos(ang), jnp.sin(ang)
    z = jnp.broadcast_to(jnp.eye(_CHUNK, dtype=jnp.complex64),
                         (2, _CHUNK, _CHUNK))
    step = 2
    while step <= _CHUNK:
        half = step // 2
        stride = _N // step
        tw = jax.lax.complex(cos_t[:, ::stride], sin_t[:, ::stride])
        xr = z.reshape(2, _CHUNK, _CHUNK // step, step)
        a = xr[:, :, :, :half]
        b = xr[:, :, :, half:]
        t = tw[:, None, None, :] * b
        z = jnp.concatenate([a + t, a - t], axis=-1).reshape(
            2, _CHUNK, _CHUNK)
        step *= 2
    tr, ti = jnp.real(z), jnp.imag(z)
    rows_a = jnp.concatenate([tr, ti], axis=2)
    rows_b = jnp.concatenate([-ti, tr], axis=2)
    m1 = jnp.stack([rows_a[0], rows_b[0]], axis=1).reshape(256, 256)
    m2 = jnp.concatenate([rows_a[1], rows_b[1]], axis=0)
    rows = []
    for step in _OUTER_STEPS:
        half = step // 2
        stride = _N // step
        rows.append(jnp.pad(cos_t[:, ::stride], ((0, 0), (0, 2048 - half))))
        rows.append(jnp.pad(sin_t[:, ::stride], ((0, 0), (0, 2048 - half))))
    rows.extend([jnp.zeros((2, 2048), jnp.float32)] * 6)
    tw_pack = jnp.stack(rows).transpose(1, 0, 2)
    return m1, m2, tw_pack[0], tw_pack[1]


def _pair_t(h_ref, tw_ref, i, c1, c2, ol):
    to = 128 * ol
    twr = tw_ref[2 * i:2 * i + 1, to:to + 128]
    twi = tw_ref[2 * i + 1:2 * i + 2, to:to + 128]
    ar = h_ref[:, c1:c1 + 128]
    ai = h_ref[:, c1 + 128:c1 + 256]
    br = h_ref[:, c2:c2 + 128]
    bi = h_ref[:, c2 + 128:c2 + 256]
    tre = twr * br - twi * bi
    tim = twr * bi + twi * br
    return ar, ai, tre, tim


def _outer_stages(h_ref, tw_ref, relu_last=False, head_only=False):
    for i, step in enumerate(_OUTER_STEPS):
        if head_only and step == 4096:
            return
        cpg = step // _CHUNK
        hc = cpg // 2
        for g in range(_N // step):
            for ol in range(hc):
                c1 = (g * cpg + ol) * 256
                c2 = c1 + hc * 256
                ar, ai, tre, tim = _pair_t(h_ref, tw_ref, i, c1, c2, ol)
                if relu_last and step == 4096:
                    h_ref[:, c1:c1 + 128] = jnp.maximum(ar + tre, 0.0)
                    h_ref[:, c1 + 128:c1 + 256] = jnp.maximum(ai + tim, 0.0)
                    h_ref[:, c2:c2 + 128] = jnp.maximum(ar - tre, 0.0)
                    h_ref[:, c2 + 128:c2 + 256] = jnp.maximum(ai - tim, 0.0)
                else:
                    h_ref[:, c1:c1 + 128] = ar + tre
                    h_ref[:, c1 + 128:c1 + 256] = ai + tim
                    if head_only and step == 2048:
                        continue
                    h_ref[:, c2:c2 + 128] = ar - tre
                    h_ref[:, c2 + 128:c2 + 256] = ai - tim


def _fwd_kernel(x_ref, m_ref, tw_ref, out_ref, h_ref):
    m1_ref, m2_ref, p_ref = (m_ref.at[:, :256], m_ref.at[:, 256:512],
                             m_ref.at[:, 512:])
    tw1_ref, tw2_ref = tw_ref.at[:16], tw_ref.at[16:]
    m1 = m1_ref[...]
    for l in range(_NCH):
        src = (l ^ (_NCH // 2)) * 256
        h_ref[:, l * 256:(l + 1) * 256] = jnp.dot(
            x_ref[:, src:src + 256], m1, preferred_element_type=jnp.float32)
    _outer_stages(h_ref, tw1_ref, relu_last=True)
    m2 = m2_ref[...]
    for l in range(_NCH // 2):
        a = h_ref[:, l * 256:(l + 1) * 256]
        b = h_ref[:, (l + 16) * 256:(l + 17) * 256]
        h_ref[:, l * 256:(l + 1) * 256] = jnp.dot(
            b, m2, preferred_element_type=jnp.float32)
        h_ref[:, (l + 16) * 256:(l + 17) * 256] = jnp.dot(
            a, m2, preferred_element_type=jnp.float32)
    _outer_stages(h_ref, tw2_ref, head_only=True)
    p = p_ref[...]
    for l in range(8):
        ar, ai, tre, tim = _pair_t(h_ref, tw2_ref, 4, l * 256,
                                   (l + 16) * 256, l)
        v = jnp.concatenate([ar + tre, ai + tim], axis=1)
        out_ref[:, l * 256:(l + 1) * 256] = jnp.dot(
            v, p, preferred_element_type=jnp.float32)


@jax.jit
def kernel(x, w1, w2):
    b = x.shape[0]
    bblk = 256 if b % 256 == 0 else b
    m1, m2, tw1, tw2 = _build_factors(w1, w2)
    j = jnp.arange(256)
    perm_cols = jnp.where(j < 128, 2 * j, 2 * (j - 128) + 1)
    p = (perm_cols[:, None] == j[None, :]).astype(jnp.float32)
    m_pack = jnp.concatenate([m1, m2, p], axis=1)
    tw_pack = jnp.concatenate([tw1, tw2], axis=0)
    return pl.pallas_call(
        _fwd_kernel,
        grid=(b // bblk,),
        in_specs=[
            pl.BlockSpec((bblk, 2 * _N), lambda i: (i, 0)),
            pl.BlockSpec((256, 768), lambda i: (0, 0)),
            pl.BlockSpec((32, 2048), lambda i: (0, 0)),
        ],
        out_specs=pl.BlockSpec((bblk, 2048), lambda i: (i, 0)),
        out_shape=jax.ShapeDtypeStruct((b, 2048), jnp.float32),
        scratch_shapes=[pltpu.VMEM((bblk, 2 * _N), jnp.float32)],
        compiler_params=pltpu.CompilerParams(
            dimension_semantics=("parallel",),
            vmem_limit_bytes=60 * 1024 * 1024,
        ),
    )(x, m_pack, tw_pack)
